# baseline (speedup 1.0000x reference)
import functools

import jax
import jax.numpy as jnp
from jax import lax
from jax.experimental import pallas as pl
from jax.experimental.pallas import tpu as pltpu
from jax.experimental.pallas import tpu_sc as plsc

N = 10000
D = 128
E = 320000
ALPHA = 0.2
RATE = 0.7

NC = 2
NS = 16
NW = NC * NS
CHUNK = 128
TOTCH = E // CHUNK
NCHK = TOTCH // NW
XW = TOTCH - NCHK * NW
NP = 10240
TROWS = NP // NS
BN = 2048
GRID = NP // BN


def _mm(x, w):
    return lax.dot_general(x, w, (((1,), (0,)), ((), ())),
                           preferred_element_type=jnp.float32,
                           precision=lax.Precision.HIGHEST)


def _lrelu(x):
    return jnp.where(x > 0.0, x, ALPHA * x)


@functools.partial(jax.jit, static_argnames=("nprops",))
def _sc_stage_call(nprops, edges, sups, zeros):
    mesh = plsc.VectorSubcoreMesh(core_axis_name="c", subcore_axis_name="s",
                                  num_cores=NC, num_subcores=NS)

    def body(*refs):
        e_refs = refs[:nprops]
        s_refs = refs[nprops:2 * nprops]
        z_ref = refs[2 * nprops]
        o_refs = refs[2 * nprops + 1: 2 * nprops + 1 + nprops]
        slab, bufs, acc_sh, isem, gsem, ssem = refs[2 * nprops + 1
                                                    + nprops:]
        cid = lax.axis_index("c")
        sid = lax.axis_index("s")
        w = cid * NS + sid
        base = w * NCHK + jnp.minimum(w, XW)

        for p in range(nprops):
            sup = s_refs[p]
            e3 = e_refs[p]

            def _g(sl, pos, bp, sup=sup):
                pltpu.async_copy(sup.at[slab.at[sl, pos, 0]], bufs.at[bp],
                                 gsem)

            def _wg(bp, sup=sup):
                pltpu.make_async_copy(sup.at[slab.at[0, 0, 0]],
                                      bufs.at[bp], gsem).wait()

            def _s(sl, pos, bp):
                pltpu.async_copy(bufs.at[bp],
                                 acc_sh.at[slab.at[sl, pos, 1]],
                                 ssem, add=True)

            def _ds():
                pltpu.make_async_copy(bufs.at[0],
                                      acc_sh.at[slab.at[0, 0, 1]],
                                      ssem).wait()

            def _idxq(k, sl, e3=e3):
                pltpu.async_copy(e3.at[pl.ds(base + 4 * k, 4)],
                                 slab.at[sl], isem)

            def _wi(e3=e3):
                pltpu.make_async_copy(e3.at[pl.ds(base, 4)],
                                      slab.at[0], isem).wait()

            pltpu.sync_copy(z_ref, acc_sh.at[pl.ds(sid * TROWS, TROWS)])
            pltpu.sync_copy(e3.at[pl.ds(base, 4)], slab.at[0])
            _idxq(1, 1)
            _g(0, 0, 0)
            plsc.subcore_barrier()

            _g(0, 1, 1)
            _wg(0)
            _s(0, 0, 0)
            _ds(); _g(0, 2, 0); _wg(1); _s(0, 1, 1)
            _ds(); _g(0, 3, 1); _wg(0); _s(0, 2, 0)
            _wi()
            _idxq(2, 2)
            _ds(); _g(1, 0, 0); _wg(1); _s(0, 3, 1)

            def steady(g, carry):
                hs = g % 3
                hn = (g + 1) % 3
                hl = (g + 2) % 3
                _ds(); _g(hs, 1, 1); _wg(0); _s(hs, 0, 0)
                _ds(); _g(hs, 2, 0); _wg(1); _s(hs, 1, 1)
                _ds(); _g(hs, 3, 1); _wg(0); _s(hs, 2, 0)
                _wi()
                _idxq(g + 2, hl)
                _ds(); _g(hn, 0, 0); _wg(1); _s(hs, 3, 1)
                return carry

            lax.fori_loop(1, 17, steady, 0)

            _ds(); _g(2, 1, 1); _wg(0); _s(2, 0, 0)
            _ds(); _g(2, 2, 0); _wg(1); _s(2, 1, 1)
            _ds(); _g(2, 3, 1); _wg(0); _s(2, 2, 0)
            _wi()
            _ds(); _g(0, 0, 0); _wg(1); _s(2, 3, 1)

            pltpu.sync_copy(e3.at[pl.ds(base + 76, 2)],
                            slab.at[1, pl.ds(0, 2)])
            exc = jnp.minimum(base + NCHK, TOTCH - 1)
            pltpu.sync_copy(e3.at[pl.ds(exc, 1)],
                            slab.at[1, pl.ds(2, 1)])

            _ds(); _g(0, 1, 1); _wg(0); _s(0, 0, 0)
            _ds(); _g(0, 2, 0); _wg(1); _s(0, 1, 1)
            _ds(); _g(0, 3, 1); _wg(0); _s(0, 2, 0)
            _ds(); _g(1, 0, 0); _wg(1); _s(0, 3, 1)

            _ds(); _g(1, 1, 1); _wg(0); _s(1, 0, 0)
            _ds(); _wg(1); _s(1, 1, 1)

            @pl.when(w < XW)
            def _extra():
                _g(1, 2, 0); _wg(0); _s(1, 2, 0)

            _ds()

            @pl.when(w < XW)
            def _extra_drain():
                _ds()

            plsc.subcore_barrier()
            pltpu.sync_copy(acc_sh.at[pl.ds(sid * TROWS, TROWS)],
                            o_refs[p].at[cid, pl.ds(sid * TROWS, TROWS)])

    kern = pl.kernel(
        body,
        out_type=tuple(jax.ShapeDtypeStruct((NC, NP, D), jnp.float32)
                       for _ in range(nprops)),
        mesh=mesh,
        scratch_types=[
            pltpu.VMEM((3, 4, 2, CHUNK), jnp.int32),
            pltpu.VMEM((2, CHUNK, D), jnp.float32),
            pltpu.VMEM_SHARED((NP, D), jnp.float32),
            pltpu.SemaphoreType.DMA,
            pltpu.SemaphoreType.DMA,
            pltpu.SemaphoreType.DMA,
        ],
    )
    return kern(*edges, *sups, zeros)


def _sc_stage(edge_list, sup_list, zeros):
    outs = _sc_stage_call(len(edge_list), tuple(edge_list), tuple(sup_list),
                          zeros)
    return list(outs)


_ROW = pl.BlockSpec((BN, D), lambda j: (j, 0))
_PART = pl.BlockSpec((NC, BN, D), lambda j: (0, j, 0))
_W = pl.BlockSpec((D, D), lambda j: (0, 0))
_W2 = pl.BlockSpec((2 * D, D), lambda j: (0, 0))
_B = pl.BlockSpec((1, D), lambda j: (0, 0))

_sds = lambda: jax.ShapeDtypeStruct((NP, D), jnp.float32)


def _tc_mm(x, w):
    def body(x_r, w_r, o_r):
        o_r[...] = _mm(x_r[...], w_r[...])

    return pl.pallas_call(
        body, grid=(GRID,),
        in_specs=[_ROW, _W],
        out_specs=_ROW,
        out_shape=_sds(),
    )(x, w)


def _tc_act_mm(part, b, w):
    def body(p_r, b_r, w_r, o_r):
        h = _lrelu(p_r[0] + p_r[1] + b_r[...])
        o_r[...] = _mm(h, w_r[...])

    return pl.pallas_call(
        body, grid=(GRID,),
        in_specs=[_PART, _B, _W],
        out_specs=_ROW,
        out_shape=_sds(),
    )(part, b.reshape(1, D), w)


def _tc_act_mm2(part, b, w1, w2):
    def body(p_r, b_r, w1_r, w2_r, o1_r, o2_r):
        h = _lrelu(p_r[0] + p_r[1] + b_r[...])
        o1_r[...] = _mm(h, w1_r[...])
        o2_r[...] = _mm(h, w2_r[...])

    return pl.pallas_call(
        body, grid=(GRID,),
        in_specs=[_PART, _B, _W, _W],
        out_specs=[_ROW, _ROW],
        out_shape=[_sds(), _sds()],
    )(part, b.reshape(1, D), w1, w2)


def _tc_u(pb, pd, b3, b4, su, tu, su_w, su_b, tu_w, tu_b):
    def body(pb_r, pd_r, b3_r, b4_r, su_r, tu_r, suw_r, sub_r, tuw_r,
             tub_r, ou_r):
        s_h2 = _lrelu(pb_r[0] + pb_r[1] + b3_r[...])
        t_h2 = _lrelu(pd_r[0] + pd_r[1] + b4_r[...])
        s_user = (_mm(s_h2, suw_r[:D]) + _mm(su_r[...], suw_r[D:])
                  + sub_r[...])
        t_user = (_mm(t_h2, tuw_r[:D]) + _mm(tu_r[...], tuw_r[D:])
                  + tub_r[...])
        ou_r[...] = (RATE * jnp.maximum(s_user, 0.0)
                     + (1.0 - RATE) * jnp.maximum(t_user, 0.0))

    return pl.pallas_call(
        body, grid=(GRID,),
        in_specs=[_PART, _PART, _B, _B, _ROW, _ROW, _W2, _B, _W2, _B],
        out_specs=_ROW,
        out_shape=_sds(),
    )(pb, pd, b3.reshape(1, D), b4.reshape(1, D), su, tu, su_w,
      su_b.reshape(1, D), tu_w, tu_b.reshape(1, D))


def _tc5(pf, pg, pi, pj, bm, bs, cm, cs, u,
         sum_w, sum_b, sus_w, sus_b, tum_w, tum_b, tus_w, tus_b):
    def body(pf_r, pg_r, pi_r, pj_r, bm_r, bs_r, cm_r, cs_r, u_r,
             sumw_r, sumb_r, susw_r, susb_r, tumw_r, tumb_r, tusw_r,
             tusb_r, om_r, os_r):
        s_m = _lrelu(pf_r[0] + pf_r[1] + bm_r[...])
        s_s = _lrelu(pg_r[0] + pg_r[1] + bs_r[...])
        t_m = _lrelu(pi_r[0] + pi_r[1] + cm_r[...])
        t_s = _lrelu(pj_r[0] + pj_r[1] + cs_r[...])
        u = u_r[...]
        sm = _mm(s_m, sumw_r[:D]) + _mm(u, sumw_r[D:]) + sumb_r[...]
        ss = _mm(s_s, susw_r[:D]) + _mm(u, susw_r[D:]) + susb_r[...]
        tm = _mm(t_m, tumw_r[:D]) + _mm(u, tumw_r[D:]) + tumb_r[...]
        ts = _mm(t_s, tusw_r[:D]) + _mm(u, tusw_r[D:]) + tusb_r[...]
        om_r[...] = RATE * sm + (1.0 - RATE) * tm
        os_r[...] = RATE * ss + (1.0 - RATE) * ts

    return pl.pallas_call(
        body, grid=(GRID,),
        in_specs=[_PART, _PART, _PART, _PART, _B, _B, _B, _B, _ROW,
                  _W2, _B, _W2, _B, _W2, _B, _W2, _B],
        out_specs=[_ROW, _ROW],
        out_shape=[_sds(), _sds()],
    )(pf, pg, pi, pj, bm.reshape(1, D), bs.reshape(1, D),
      cm.reshape(1, D), cs.reshape(1, D), u,
      sum_w, sum_b.reshape(1, D), sus_w, sus_b.reshape(1, D),
      tum_w, tum_b.reshape(1, D), tus_w, tus_b.reshape(1, D))


def kernel(source_ufea, target_ufea, source_UV_edge_index,
           source_VU_edge_index, target_UV_edge_index, target_VU_edge_index,
           l1_gc1_W, l1_gc1_b, l1_gc2_W, l1_gc2_b, l1_gc3_W, l1_gc3_b,
           l1_gc4_W, l1_gc4_b, l1_su_W, l1_su_b, l1_tu_W, l1_tu_b,
           l2_gc1_W, l2_gc1_b, l2_gc2_W, l2_gc2_b, l2_gc3m_W, l2_gc3m_b,
           l2_gc3s_W, l2_gc3s_b, l2_gc4m_W, l2_gc4m_b, l2_gc4s_W, l2_gc4s_b,
           l2_sum_W, l2_sum_b, l2_sus_W, l2_sus_b, l2_tum_W, l2_tum_b,
           l2_tus_W, l2_tus_b):
    su = jnp.pad(source_ufea, ((0, NP - N), (0, 0)))
    tu = jnp.pad(target_ufea, ((0, NP - N), (0, 0)))
    def _chunked(e):
        return e.reshape(2, TOTCH, CHUNK).transpose(1, 0, 2)

    e_suv = _chunked(source_UV_edge_index)
    e_svu = _chunked(source_VU_edge_index)
    e_tuv = _chunked(target_UV_edge_index)
    e_tvu = _chunked(target_VU_edge_index)
    zeros = jnp.zeros((TROWS, D), jnp.float32)

    sup_a = _tc_mm(su, l1_gc1_W)
    part_a, = _sc_stage([e_svu], [sup_a], zeros)
    sup_c = _tc_mm(tu, l1_gc2_W)
    part_c, = _sc_stage([e_tvu], [sup_c], zeros)
    sup_b = _tc_act_mm(part_a, l1_gc1_b, l1_gc3_W)
    part_b, = _sc_stage([e_suv], [sup_b], zeros)
    sup_d = _tc_act_mm(part_c, l1_gc2_b, l1_gc4_W)
    part_d, = _sc_stage([e_tuv], [sup_d], zeros)
    u = _tc_u(part_b, part_d, l1_gc3_b, l1_gc4_b, su, tu,
              l1_su_W, l1_su_b, l1_tu_W, l1_tu_b)
    sup_e = _tc_mm(u, l2_gc1_W)
    part_e, = _sc_stage([e_svu], [sup_e], zeros)
    sup_h = _tc_mm(u, l2_gc2_W)
    part_h, = _sc_stage([e_tvu], [sup_h], zeros)
    sup_f, sup_g = _tc_act_mm2(part_e, l2_gc1_b, l2_gc3m_W, l2_gc3s_W)
    part_f, = _sc_stage([e_suv], [sup_f], zeros)
    sup_i, sup_j = _tc_act_mm2(part_h, l2_gc2_b, l2_gc4m_W, l2_gc4s_W)
    part_i, = _sc_stage([e_tuv], [sup_i], zeros)
    part_g, = _sc_stage([e_suv], [sup_g], zeros)
    part_j, = _sc_stage([e_tuv], [sup_j], zeros)
    mean, sigma = _tc5(part_f, part_g, part_i, part_j,
                       l2_gc3m_b, l2_gc3s_b, l2_gc4m_b, l2_gc4s_b, u,
                       l2_sum_W, l2_sum_b, l2_sus_W, l2_sus_b,
                       l2_tum_W, l2_tum_b, l2_tus_W, l2_tus_b)
    return (mean[:N], sigma[:N])


# speedup vs baseline: 1.0280x; 1.0280x over previous
import functools

import jax
import jax.numpy as jnp
from jax import lax
from jax.experimental import pallas as pl
from jax.experimental.pallas import tpu as pltpu
from jax.experimental.pallas import tpu_sc as plsc

N = 10000
D = 128
E = 320000
ALPHA = 0.2
RATE = 0.7

NC = 2
NS = 16
NW = NC * NS
CHUNK = 128
TOTCH = E // CHUNK
NCHK = TOTCH // NW
XW = TOTCH - NCHK * NW
NP = 10240
TROWS = NP // NS
BN = 2048
GRID = NP // BN


def _mm(x, w):
    return lax.dot_general(x, w, (((1,), (0,)), ((), ())),
                           preferred_element_type=jnp.float32,
                           precision=lax.Precision.HIGHEST)


def _lrelu(x):
    return jnp.where(x > 0.0, x, ALPHA * x)


@functools.partial(jax.jit, static_argnames=("nprops",))
def _sc_stage_call(nprops, edges, sups, zeros):
    mesh = plsc.VectorSubcoreMesh(core_axis_name="c", subcore_axis_name="s",
                                  num_cores=NC, num_subcores=NS)

    def body(*refs):
        e_refs = refs[:nprops]
        s_refs = refs[nprops:2 * nprops]
        z_ref = refs[2 * nprops]
        o_refs = refs[2 * nprops + 1: 2 * nprops + 1 + nprops]
        slab, bufs, acc_sh, isem, gsem, ssem = refs[2 * nprops + 1
                                                    + nprops:]
        cid = lax.axis_index("c")
        sid = lax.axis_index("s")
        w = cid * NS + sid
        base = w * NCHK + jnp.minimum(w, XW)

        for p in range(nprops):
            sup = s_refs[p]
            e3 = e_refs[p]

            def _g(sl, pos, bp, sup=sup):
                pltpu.async_copy(sup.at[slab.at[sl, pos, 0, pl.ds(0, 64)]],
                                 bufs.at[bp, pl.ds(0, 64)], gsem)
                pltpu.async_copy(sup.at[slab.at[sl, pos, 0,
                                                pl.ds(64, 64)]],
                                 bufs.at[bp, pl.ds(64, 64)], gsem)

            def _wg(bp, sup=sup):
                pltpu.make_async_copy(sup.at[slab.at[0, 0, 0]],
                                      bufs.at[bp], gsem).wait()

            def _s(sl, pos, bp):
                pltpu.async_copy(bufs.at[bp],
                                 acc_sh.at[slab.at[sl, pos, 1]],
                                 ssem, add=True)

            def _ds():
                pltpu.make_async_copy(bufs.at[0],
                                      acc_sh.at[slab.at[0, 0, 1]],
                                      ssem).wait()

            def _idxq(k, sl, e3=e3):
                pltpu.async_copy(e3.at[pl.ds(base + 4 * k, 4)],
                                 slab.at[sl], isem)

            def _wi(e3=e3):
                pltpu.make_async_copy(e3.at[pl.ds(base, 4)],
                                      slab.at[0], isem).wait()

            pltpu.sync_copy(z_ref, acc_sh.at[pl.ds(sid * TROWS, TROWS)])
            pltpu.sync_copy(e3.at[pl.ds(base, 4)], slab.at[0])
            _idxq(1, 1)
            _g(0, 0, 0)
            plsc.subcore_barrier()

            _g(0, 1, 1)
            _wg(0)
            _s(0, 0, 0)
            _ds(); _g(0, 2, 0); _wg(1); _s(0, 1, 1)
            _ds(); _g(0, 3, 1); _wg(0); _s(0, 2, 0)
            _wi()
            _idxq(2, 2)
            _ds(); _g(1, 0, 0); _wg(1); _s(0, 3, 1)

            def steady(g, carry):
                hs = g % 3
                hn = (g + 1) % 3
                hl = (g + 2) % 3
                _ds(); _g(hs, 1, 1); _wg(0); _s(hs, 0, 0)
                _ds(); _g(hs, 2, 0); _wg(1); _s(hs, 1, 1)
                _ds(); _g(hs, 3, 1); _wg(0); _s(hs, 2, 0)
                _wi()
                _idxq(g + 2, hl)
                _ds(); _g(hn, 0, 0); _wg(1); _s(hs, 3, 1)
                return carry

            lax.fori_loop(1, 17, steady, 0)

            _ds(); _g(2, 1, 1); _wg(0); _s(2, 0, 0)
            _ds(); _g(2, 2, 0); _wg(1); _s(2, 1, 1)
            _ds(); _g(2, 3, 1); _wg(0); _s(2, 2, 0)
            _wi()
            _ds(); _g(0, 0, 0); _wg(1); _s(2, 3, 1)

            pltpu.sync_copy(e3.at[pl.ds(base + 76, 2)],
                            slab.at[1, pl.ds(0, 2)])
            exc = jnp.minimum(base + NCHK, TOTCH - 1)
            pltpu.sync_copy(e3.at[pl.ds(exc, 1)],
                            slab.at[1, pl.ds(2, 1)])

            _ds(); _g(0, 1, 1); _wg(0); _s(0, 0, 0)
            _ds(); _g(0, 2, 0); _wg(1); _s(0, 1, 1)
            _ds(); _g(0, 3, 1); _wg(0); _s(0, 2, 0)
            _ds(); _g(1, 0, 0); _wg(1); _s(0, 3, 1)

            _ds(); _g(1, 1, 1); _wg(0); _s(1, 0, 0)
            _ds(); _wg(1); _s(1, 1, 1)

            @pl.when(w < XW)
            def _extra():
                _g(1, 2, 0); _wg(0); _s(1, 2, 0)

            _ds()

            @pl.when(w < XW)
            def _extra_drain():
                _ds()

            plsc.subcore_barrier()
            pltpu.sync_copy(acc_sh.at[pl.ds(sid * TROWS, TROWS)],
                            o_refs[p].at[cid, pl.ds(sid * TROWS, TROWS)])

    kern = pl.kernel(
        body,
        out_type=tuple(jax.ShapeDtypeStruct((NC, NP, D), jnp.float32)
                       for _ in range(nprops)),
        mesh=mesh,
        scratch_types=[
            pltpu.VMEM((3, 4, 2, CHUNK), jnp.int32),
            pltpu.VMEM((2, CHUNK, D), jnp.float32),
            pltpu.VMEM_SHARED((NP, D), jnp.float32),
            pltpu.SemaphoreType.DMA,
            pltpu.SemaphoreType.DMA,
            pltpu.SemaphoreType.DMA,
        ],
    )
    return kern(*edges, *sups, zeros)


def _sc_stage(edge_list, sup_list, zeros):
    outs = _sc_stage_call(len(edge_list), tuple(edge_list), tuple(sup_list),
                          zeros)
    return list(outs)


_ROW = pl.BlockSpec((BN, D), lambda j: (j, 0))
_PART = pl.BlockSpec((NC, BN, D), lambda j: (0, j, 0))
_W = pl.BlockSpec((D, D), lambda j: (0, 0))
_W2 = pl.BlockSpec((2 * D, D), lambda j: (0, 0))
_B = pl.BlockSpec((1, D), lambda j: (0, 0))

_sds = lambda: jax.ShapeDtypeStruct((NP, D), jnp.float32)


def _tc1(su, tu, w1, w2):
    def body(su_r, tu_r, w1_r, w2_r, oa_r, oc_r):
        oa_r[...] = _mm(su_r[...], w1_r[...])
        oc_r[...] = _mm(tu_r[...], w2_r[...])

    return pl.pallas_call(
        body, grid=(GRID,),
        in_specs=[_ROW, _ROW, _W, _W],
        out_specs=[_ROW, _ROW],
        out_shape=[_sds(), _sds()],
    )(su, tu, w1, w2)


def _tc2(pa, pc, b1, b2, w3, w4):
    def body(pa_r, pc_r, b1_r, b2_r, w3_r, w4_r, ob_r, od_r):
        s_h1 = _lrelu(pa_r[0] + pa_r[1] + b1_r[...])
        t_h1 = _lrelu(pc_r[0] + pc_r[1] + b2_r[...])
        ob_r[...] = _mm(s_h1, w3_r[...])
        od_r[...] = _mm(t_h1, w4_r[...])

    return pl.pallas_call(
        body, grid=(GRID,),
        in_specs=[_PART, _PART, _B, _B, _W, _W],
        out_specs=[_ROW, _ROW],
        out_shape=[_sds(), _sds()],
    )(pa, pc, b1.reshape(1, D), b2.reshape(1, D), w3, w4)


def _tc3(pb, pd, b3, b4, su, tu, su_w, su_b, tu_w, tu_b, we, wh):
    def body(pb_r, pd_r, b3_r, b4_r, su_r, tu_r, suw_r, sub_r, tuw_r,
             tub_r, we_r, wh_r, ou_r, oe_r, oh_r):
        s_h2 = _lrelu(pb_r[0] + pb_r[1] + b3_r[...])
        t_h2 = _lrelu(pd_r[0] + pd_r[1] + b4_r[...])
        s_user = (_mm(s_h2, suw_r[:D]) + _mm(su_r[...], suw_r[D:])
                  + sub_r[...])
        t_user = (_mm(t_h2, tuw_r[:D]) + _mm(tu_r[...], tuw_r[D:])
                  + tub_r[...])
        u = (RATE * jnp.maximum(s_user, 0.0)
             + (1.0 - RATE) * jnp.maximum(t_user, 0.0))
        ou_r[...] = u
        oe_r[...] = _mm(u, we_r[...])
        oh_r[...] = _mm(u, wh_r[...])

    return pl.pallas_call(
        body, grid=(GRID,),
        in_specs=[_PART, _PART, _B, _B, _ROW, _ROW, _W2, _B, _W2, _B,
                  _W, _W],
        out_specs=[_ROW, _ROW, _ROW],
        out_shape=[_sds(), _sds(), _sds()],
    )(pb, pd, b3.reshape(1, D), b4.reshape(1, D), su, tu, su_w,
      su_b.reshape(1, D), tu_w, tu_b.reshape(1, D), we, wh)


def _tc4(ps, pt, b1, b2, w3m, w3s, w4m, w4s):
    def body(ps_r, pt_r, b1_r, b2_r, w3m_r, w3s_r, w4m_r, w4s_r,
             of_r, og_r, oi_r, oj_r):
        s_g1 = _lrelu(ps_r[0] + ps_r[1] + b1_r[...])
        t_g1 = _lrelu(pt_r[0] + pt_r[1] + b2_r[...])
        of_r[...] = _mm(s_g1, w3m_r[...])
        og_r[...] = _mm(s_g1, w3s_r[...])
        oi_r[...] = _mm(t_g1, w4m_r[...])
        oj_r[...] = _mm(t_g1, w4s_r[...])

    return pl.pallas_call(
        body, grid=(GRID,),
        in_specs=[_PART, _PART, _B, _B, _W, _W, _W, _W],
        out_specs=[_ROW, _ROW, _ROW, _ROW],
        out_shape=[_sds(), _sds(), _sds(), _sds()],
    )(ps, pt, b1.reshape(1, D), b2.reshape(1, D), w3m, w3s, w4m, w4s)


def _tc5(pf, pg, pi, pj, bm, bs, cm, cs, u,
         sum_w, sum_b, sus_w, sus_b, tum_w, tum_b, tus_w, tus_b):
    def body(pf_r, pg_r, pi_r, pj_r, bm_r, bs_r, cm_r, cs_r, u_r,
             sumw_r, sumb_r, susw_r, susb_r, tumw_r, tumb_r, tusw_r,
             tusb_r, om_r, os_r):
        s_m = _lrelu(pf_r[0] + pf_r[1] + bm_r[...])
        s_s = _lrelu(pg_r[0] + pg_r[1] + bs_r[...])
        t_m = _lrelu(pi_r[0] + pi_r[1] + cm_r[...])
        t_s = _lrelu(pj_r[0] + pj_r[1] + cs_r[...])
        u = u_r[...]
        sm = _mm(s_m, sumw_r[:D]) + _mm(u, sumw_r[D:]) + sumb_r[...]
        ss = _mm(s_s, susw_r[:D]) + _mm(u, susw_r[D:]) + susb_r[...]
        tm = _mm(t_m, tumw_r[:D]) + _mm(u, tumw_r[D:]) + tumb_r[...]
        ts = _mm(t_s, tusw_r[:D]) + _mm(u, tusw_r[D:]) + tusb_r[...]
        om_r[...] = RATE * sm + (1.0 - RATE) * tm
        os_r[...] = RATE * ss + (1.0 - RATE) * ts

    return pl.pallas_call(
        body, grid=(GRID,),
        in_specs=[_PART, _PART, _PART, _PART, _B, _B, _B, _B, _ROW,
                  _W2, _B, _W2, _B, _W2, _B, _W2, _B],
        out_specs=[_ROW, _ROW],
        out_shape=[_sds(), _sds()],
    )(pf, pg, pi, pj, bm.reshape(1, D), bs.reshape(1, D),
      cm.reshape(1, D), cs.reshape(1, D), u,
      sum_w, sum_b.reshape(1, D), sus_w, sus_b.reshape(1, D),
      tum_w, tum_b.reshape(1, D), tus_w, tus_b.reshape(1, D))


def kernel(source_ufea, target_ufea, source_UV_edge_index,
           source_VU_edge_index, target_UV_edge_index, target_VU_edge_index,
           l1_gc1_W, l1_gc1_b, l1_gc2_W, l1_gc2_b, l1_gc3_W, l1_gc3_b,
           l1_gc4_W, l1_gc4_b, l1_su_W, l1_su_b, l1_tu_W, l1_tu_b,
           l2_gc1_W, l2_gc1_b, l2_gc2_W, l2_gc2_b, l2_gc3m_W, l2_gc3m_b,
           l2_gc3s_W, l2_gc3s_b, l2_gc4m_W, l2_gc4m_b, l2_gc4s_W, l2_gc4s_b,
           l2_sum_W, l2_sum_b, l2_sus_W, l2_sus_b, l2_tum_W, l2_tum_b,
           l2_tus_W, l2_tus_b):
    su = jnp.pad(source_ufea, ((0, NP - N), (0, 0)))
    tu = jnp.pad(target_ufea, ((0, NP - N), (0, 0)))
    def _chunked(e):
        return e.reshape(2, TOTCH, CHUNK).transpose(1, 0, 2)

    e_suv = _chunked(source_UV_edge_index)
    e_svu = _chunked(source_VU_edge_index)
    e_tuv = _chunked(target_UV_edge_index)
    e_tvu = _chunked(target_VU_edge_index)
    zeros = jnp.zeros((TROWS, D), jnp.float32)

    sup_a, sup_c = _tc1(su, tu, l1_gc1_W, l1_gc2_W)
    part_a, part_c = _sc_stage([e_svu, e_tvu], [sup_a, sup_c], zeros)
    sup_b, sup_d = _tc2(part_a, part_c, l1_gc1_b, l1_gc2_b,
                        l1_gc3_W, l1_gc4_W)
    part_b, part_d = _sc_stage([e_suv, e_tuv], [sup_b, sup_d], zeros)
    u, sup_e, sup_h = _tc3(part_b, part_d, l1_gc3_b, l1_gc4_b, su, tu,
                           l1_su_W, l1_su_b, l1_tu_W, l1_tu_b,
                           l2_gc1_W, l2_gc2_W)
    part_e, part_h = _sc_stage([e_svu, e_tvu], [sup_e, sup_h], zeros)
    sup_f, sup_g, sup_i, sup_j = _tc4(part_e, part_h, l2_gc1_b, l2_gc2_b,
                                      l2_gc3m_W, l2_gc3s_W,
                                      l2_gc4m_W, l2_gc4s_W)
    part_f, part_g, part_i, part_j = _sc_stage(
        [e_suv, e_suv, e_tuv, e_tuv], [sup_f, sup_g, sup_i, sup_j], zeros)
    mean, sigma = _tc5(part_f, part_g, part_i, part_j,
                       l2_gc3m_b, l2_gc3s_b, l2_gc4m_b, l2_gc4s_b, u,
                       l2_sum_W, l2_sum_b, l2_sus_W, l2_sus_b,
                       l2_tum_W, l2_tum_b, l2_tus_W, l2_tus_b)
    return (mean[:N], sigma[:N])


# speedup vs baseline: 1.0498x; 1.0212x over previous
import functools

import jax
import jax.numpy as jnp
from jax import lax
from jax.experimental import pallas as pl
from jax.experimental.pallas import tpu as pltpu
from jax.experimental.pallas import tpu_sc as plsc

N = 10000
D = 128
E = 320000
ALPHA = 0.2
RATE = 0.7

NC = 2
NS = 16
NW = NC * NS
CHUNK = 128
TOTCH = E // CHUNK
NCHK = TOTCH // NW
XW = TOTCH - NCHK * NW
NP = 10240
TROWS = NP // NS
BN = 2048
GRID = NP // BN


def _mm(x, w):
    return lax.dot_general(x, w, (((1,), (0,)), ((), ())),
                           preferred_element_type=jnp.float32,
                           precision=lax.Precision.HIGHEST)


def _lrelu(x):
    return jnp.where(x > 0.0, x, ALPHA * x)


@functools.partial(jax.jit, static_argnames=("nprops",))
def _sc_stage_call(nprops, edges, sups, zeros):
    mesh = plsc.VectorSubcoreMesh(core_axis_name="c", subcore_axis_name="s",
                                  num_cores=NC, num_subcores=NS)

    def body(*refs):
        e_refs = refs[:nprops]
        s_refs = refs[nprops:2 * nprops]
        z_ref = refs[2 * nprops]
        o_refs = refs[2 * nprops + 1: 2 * nprops + 1 + nprops]
        slab, bufs, acc_sh, isem, gsem, ssem = refs[2 * nprops + 1
                                                    + nprops:]
        cid = lax.axis_index("c")
        sid = lax.axis_index("s")
        w = cid * NS + sid
        base = w * NCHK + jnp.minimum(w, XW)

        for p in range(nprops):
            sup = s_refs[p]
            e3 = e_refs[p]

            def _g(sl, pos, bp, sup=sup):
                pltpu.async_copy(sup.at[slab.at[sl, pos, 0]], bufs.at[bp],
                                 gsem)

            def _wg(bp, sup=sup):
                pltpu.make_async_copy(sup.at[slab.at[0, 0, 0]],
                                      bufs.at[bp], gsem).wait()

            def _s(sl, pos, bp):
                pltpu.async_copy(bufs.at[bp],
                                 acc_sh.at[slab.at[sl, pos, 1]],
                                 ssem, add=True)

            def _ds():
                pltpu.make_async_copy(bufs.at[0],
                                      acc_sh.at[slab.at[0, 0, 1]],
                                      ssem).wait()

            def _idxq(k, sl, e3=e3):
                pltpu.async_copy(e3.at[pl.ds(base + 4 * k, 4)],
                                 slab.at[sl], isem)

            def _wi(e3=e3):
                pltpu.make_async_copy(e3.at[pl.ds(base, 4)],
                                      slab.at[0], isem).wait()

            pltpu.sync_copy(z_ref.at[pl.ds(sid * TROWS, TROWS)],
                            acc_sh.at[pl.ds(sid * TROWS, TROWS)])
            pltpu.sync_copy(e3.at[pl.ds(base, 4)], slab.at[0])
            _idxq(1, 1)
            _g(0, 0, 0)
            plsc.subcore_barrier()

            _g(0, 1, 1)
            _wg(0)
            _s(0, 0, 0)
            _ds(); _g(0, 2, 0); _wg(1); _s(0, 1, 1)
            _ds(); _g(0, 3, 1); _wg(0); _s(0, 2, 0)
            _wi()
            _idxq(2, 2)
            _ds(); _g(1, 0, 0); _wg(1); _s(0, 3, 1)

            def steady(g, carry):
                hs = g % 3
                hn = (g + 1) % 3
                hl = (g + 2) % 3
                _ds(); _g(hs, 1, 1); _wg(0); _s(hs, 0, 0)
                _ds(); _g(hs, 2, 0); _wg(1); _s(hs, 1, 1)
                _ds(); _g(hs, 3, 1); _wg(0); _s(hs, 2, 0)
                _wi()
                _idxq(g + 2, hl)
                _ds(); _g(hn, 0, 0); _wg(1); _s(hs, 3, 1)
                return carry

            lax.fori_loop(1, 17, steady, 0)

            _ds(); _g(2, 1, 1); _wg(0); _s(2, 0, 0)
            _ds(); _g(2, 2, 0); _wg(1); _s(2, 1, 1)
            _ds(); _g(2, 3, 1); _wg(0); _s(2, 2, 0)
            _wi()
            _ds(); _g(0, 0, 0); _wg(1); _s(2, 3, 1)

            pltpu.sync_copy(e3.at[pl.ds(base + 76, 2)],
                            slab.at[1, pl.ds(0, 2)])
            exc = jnp.minimum(base + NCHK, TOTCH - 1)
            pltpu.sync_copy(e3.at[pl.ds(exc, 1)],
                            slab.at[1, pl.ds(2, 1)])

            _ds(); _g(0, 1, 1); _wg(0); _s(0, 0, 0)
            _ds(); _g(0, 2, 0); _wg(1); _s(0, 1, 1)
            _ds(); _g(0, 3, 1); _wg(0); _s(0, 2, 0)
            _ds(); _g(1, 0, 0); _wg(1); _s(0, 3, 1)

            _ds(); _g(1, 1, 1); _wg(0); _s(1, 0, 0)
            _ds(); _wg(1); _s(1, 1, 1)

            @pl.when(w < XW)
            def _extra():
                _g(1, 2, 0); _wg(0); _s(1, 2, 0)

            _ds()

            @pl.when(w < XW)
            def _extra_drain():
                _ds()

            plsc.subcore_barrier()
            pltpu.sync_copy(acc_sh.at[pl.ds(sid * TROWS, TROWS)],
                            o_refs[p].at[cid, pl.ds(sid * TROWS, TROWS)])

    kern = pl.kernel(
        body,
        out_type=tuple(jax.ShapeDtypeStruct((NC, NP, D), jnp.float32)
                       for _ in range(nprops)),
        mesh=mesh,
        scratch_types=[
            pltpu.VMEM((3, 4, 2, CHUNK), jnp.int32),
            pltpu.VMEM((2, CHUNK, D), jnp.float32),
            pltpu.VMEM_SHARED((NP, D), jnp.float32),
            pltpu.SemaphoreType.DMA,
            pltpu.SemaphoreType.DMA,
            pltpu.SemaphoreType.DMA,
        ],
    )
    return kern(*edges, *sups, zeros)


def _sc_stage(edge_list, sup_list, zeros):
    outs = _sc_stage_call(len(edge_list), tuple(edge_list), tuple(sup_list),
                          zeros)
    return list(outs)


_ROW = pl.BlockSpec((BN, D), lambda j: (j, 0))
_PART = pl.BlockSpec((NC, BN, D), lambda j: (0, j, 0))
_W = pl.BlockSpec((D, D), lambda j: (0, 0))
_W2 = pl.BlockSpec((2 * D, D), lambda j: (0, 0))
_B = pl.BlockSpec((1, D), lambda j: (0, 0))

_sds = lambda: jax.ShapeDtypeStruct((NP, D), jnp.float32)


def _tc1(su, tu, w1, w2):
    def body(su_r, tu_r, w1_r, w2_r, oa_r, oc_r):
        oa_r[...] = _mm(su_r[...], w1_r[...])
        oc_r[...] = _mm(tu_r[...], w2_r[...])

    return pl.pallas_call(
        body, grid=(GRID,),
        in_specs=[_ROW, _ROW, _W, _W],
        out_specs=[_ROW, _ROW],
        out_shape=[_sds(), _sds()],
    )(su, tu, w1, w2)


def _tc2(pa, pc, b1, b2, w3, w4):
    def body(pa_r, pc_r, b1_r, b2_r, w3_r, w4_r, ob_r, od_r):
        s_h1 = _lrelu(pa_r[0] + pa_r[1] + b1_r[...])
        t_h1 = _lrelu(pc_r[0] + pc_r[1] + b2_r[...])
        ob_r[...] = _mm(s_h1, w3_r[...])
        od_r[...] = _mm(t_h1, w4_r[...])

    return pl.pallas_call(
        body, grid=(GRID,),
        in_specs=[_PART, _PART, _B, _B, _W, _W],
        out_specs=[_ROW, _ROW],
        out_shape=[_sds(), _sds()],
    )(pa, pc, b1.reshape(1, D), b2.reshape(1, D), w3, w4)


def _tc3(pb, pd, b3, b4, su, tu, su_w, su_b, tu_w, tu_b, we, wh):
    def body(pb_r, pd_r, b3_r, b4_r, su_r, tu_r, suw_r, sub_r, tuw_r,
             tub_r, we_r, wh_r, ou_r, oe_r, oh_r):
        s_h2 = _lrelu(pb_r[0] + pb_r[1] + b3_r[...])
        t_h2 = _lrelu(pd_r[0] + pd_r[1] + b4_r[...])
        s_user = (_mm(s_h2, suw_r[:D]) + _mm(su_r[...], suw_r[D:])
                  + sub_r[...])
        t_user = (_mm(t_h2, tuw_r[:D]) + _mm(tu_r[...], tuw_r[D:])
                  + tub_r[...])
        u = (RATE * jnp.maximum(s_user, 0.0)
             + (1.0 - RATE) * jnp.maximum(t_user, 0.0))
        ou_r[...] = u
        oe_r[...] = _mm(u, we_r[...])
        oh_r[...] = _mm(u, wh_r[...])

    return pl.pallas_call(
        body, grid=(GRID,),
        in_specs=[_PART, _PART, _B, _B, _ROW, _ROW, _W2, _B, _W2, _B,
                  _W, _W],
        out_specs=[_ROW, _ROW, _ROW],
        out_shape=[_sds(), _sds(), _sds()],
    )(pb, pd, b3.reshape(1, D), b4.reshape(1, D), su, tu, su_w,
      su_b.reshape(1, D), tu_w, tu_b.reshape(1, D), we, wh)


def _tc4(ps, pt, b1, b2, w3m, w3s, w4m, w4s):
    def body(ps_r, pt_r, b1_r, b2_r, w3m_r, w3s_r, w4m_r, w4s_r,
             of_r, og_r, oi_r, oj_r):
        s_g1 = _lrelu(ps_r[0] + ps_r[1] + b1_r[...])
        t_g1 = _lrelu(pt_r[0] + pt_r[1] + b2_r[...])
        of_r[...] = _mm(s_g1, w3m_r[...])
        og_r[...] = _mm(s_g1, w3s_r[...])
        oi_r[...] = _mm(t_g1, w4m_r[...])
        oj_r[...] = _mm(t_g1, w4s_r[...])

    return pl.pallas_call(
        body, grid=(GRID,),
        in_specs=[_PART, _PART, _B, _B, _W, _W, _W, _W],
        out_specs=[_ROW, _ROW, _ROW, _ROW],
        out_shape=[_sds(), _sds(), _sds(), _sds()],
    )(ps, pt, b1.reshape(1, D), b2.reshape(1, D), w3m, w3s, w4m, w4s)


def _tc5(pf, pg, pi, pj, bm, bs, cm, cs, u,
         sum_w, sum_b, sus_w, sus_b, tum_w, tum_b, tus_w, tus_b):
    def body(pf_r, pg_r, pi_r, pj_r, bm_r, bs_r, cm_r, cs_r, u_r,
             sumw_r, sumb_r, susw_r, susb_r, tumw_r, tumb_r, tusw_r,
             tusb_r, om_r, os_r):
        s_m = _lrelu(pf_r[0] + pf_r[1] + bm_r[...])
        s_s = _lrelu(pg_r[0] + pg_r[1] + bs_r[...])
        t_m = _lrelu(pi_r[0] + pi_r[1] + cm_r[...])
        t_s = _lrelu(pj_r[0] + pj_r[1] + cs_r[...])
        u = u_r[...]
        sm = _mm(s_m, sumw_r[:D]) + _mm(u, sumw_r[D:]) + sumb_r[...]
        ss = _mm(s_s, susw_r[:D]) + _mm(u, susw_r[D:]) + susb_r[...]
        tm = _mm(t_m, tumw_r[:D]) + _mm(u, tumw_r[D:]) + tumb_r[...]
        ts = _mm(t_s, tusw_r[:D]) + _mm(u, tusw_r[D:]) + tusb_r[...]
        om_r[...] = RATE * sm + (1.0 - RATE) * tm
        os_r[...] = RATE * ss + (1.0 - RATE) * ts

    rowN = pl.BlockSpec((N // GRID, D), lambda j: (j, 0))
    partN = pl.BlockSpec((NC, N // GRID, D), lambda j: (0, j, 0))
    sdsN = jax.ShapeDtypeStruct((N, D), jnp.float32)
    return pl.pallas_call(
        body, grid=(GRID,),
        in_specs=[partN, partN, partN, partN, _B, _B, _B, _B, rowN,
                  _W2, _B, _W2, _B, _W2, _B, _W2, _B],
        out_specs=[rowN, rowN],
        out_shape=[sdsN, sdsN],
    )(pf, pg, pi, pj, bm.reshape(1, D), bs.reshape(1, D),
      cm.reshape(1, D), cs.reshape(1, D), u,
      sum_w, sum_b.reshape(1, D), sus_w, sus_b.reshape(1, D),
      tum_w, tum_b.reshape(1, D), tus_w, tus_b.reshape(1, D))


def kernel(source_ufea, target_ufea, source_UV_edge_index,
           source_VU_edge_index, target_UV_edge_index, target_VU_edge_index,
           l1_gc1_W, l1_gc1_b, l1_gc2_W, l1_gc2_b, l1_gc3_W, l1_gc3_b,
           l1_gc4_W, l1_gc4_b, l1_su_W, l1_su_b, l1_tu_W, l1_tu_b,
           l2_gc1_W, l2_gc1_b, l2_gc2_W, l2_gc2_b, l2_gc3m_W, l2_gc3m_b,
           l2_gc3s_W, l2_gc3s_b, l2_gc4m_W, l2_gc4m_b, l2_gc4s_W, l2_gc4s_b,
           l2_sum_W, l2_sum_b, l2_sus_W, l2_sus_b, l2_tum_W, l2_tum_b,
           l2_tus_W, l2_tus_b):
    su = jnp.pad(source_ufea, ((0, NP - N), (0, 0)))
    tu = jnp.pad(target_ufea, ((0, NP - N), (0, 0)))
    def _chunked(e):
        return e.reshape(2, TOTCH, CHUNK).transpose(1, 0, 2)

    e_suv = _chunked(source_UV_edge_index)
    e_svu = _chunked(source_VU_edge_index)
    e_tuv = _chunked(target_UV_edge_index)
    e_tvu = _chunked(target_VU_edge_index)
    zeros = jnp.zeros((NP, D), jnp.float32)

    sup_a, sup_c = _tc1(su, tu, l1_gc1_W, l1_gc2_W)
    part_a, part_c = _sc_stage([e_svu, e_tvu], [sup_a, sup_c], zeros)
    sup_b, sup_d = _tc2(part_a, part_c, l1_gc1_b, l1_gc2_b,
                        l1_gc3_W, l1_gc4_W)
    part_b, part_d = _sc_stage([e_suv, e_tuv], [sup_b, sup_d], zeros)
    u, sup_e, sup_h = _tc3(part_b, part_d, l1_gc3_b, l1_gc4_b, su, tu,
                           l1_su_W, l1_su_b, l1_tu_W, l1_tu_b,
                           l2_gc1_W, l2_gc2_W)
    part_e, part_h = _sc_stage([e_svu, e_tvu], [sup_e, sup_h], zeros)
    sup_f, sup_g, sup_i, sup_j = _tc4(part_e, part_h, l2_gc1_b, l2_gc2_b,
                                      l2_gc3m_W, l2_gc3s_W,
                                      l2_gc4m_W, l2_gc4s_W)
    part_f, part_g, part_i, part_j = _sc_stage(
        [e_suv, e_suv, e_tuv, e_tuv], [sup_f, sup_g, sup_i, sup_j], zeros)
    mean, sigma = _tc5(part_f, part_g, part_i, part_j,
                       l2_gc3m_b, l2_gc3s_b, l2_gc4m_b, l2_gc4s_b, u,
                       l2_sum_W, l2_sum_b, l2_sus_W, l2_sus_b,
                       l2_tum_W, l2_tum_b, l2_tus_W, l2_tus_b)
    return (mean, sigma)


# speedup vs baseline: 1.0529x; 1.0030x over previous
import functools

import jax
import jax.numpy as jnp
from jax import lax
from jax.experimental import pallas as pl
from jax.experimental.pallas import tpu as pltpu
from jax.experimental.pallas import tpu_sc as plsc

N = 10000
D = 128
E = 320000
ALPHA = 0.2
RATE = 0.7

NC = 2
NS = 16
NW = NC * NS
CHUNK = 128
TOTCH = E // CHUNK
NCHK = TOTCH // NW
XW = TOTCH - NCHK * NW
NP = 10240
TROWS = NP // NS
BN = 2048
GRID = NP // BN


def _mm(x, w):
    return lax.dot_general(x, w, (((1,), (0,)), ((), ())),
                           preferred_element_type=jnp.float32,
                           precision=lax.Precision.HIGHEST)


def _lrelu(x):
    return jnp.where(x > 0.0, x, ALPHA * x)


@functools.partial(jax.jit, static_argnames=("nprops",))
def _sc_stage_call(nprops, edges, sups, zeros):
    mesh = plsc.VectorSubcoreMesh(core_axis_name="c", subcore_axis_name="s",
                                  num_cores=NC, num_subcores=NS)

    def body(*refs):
        e_refs = refs[:nprops]
        s_refs = refs[nprops:2 * nprops]
        z_ref = refs[2 * nprops]
        o_refs = refs[2 * nprops + 1: 2 * nprops + 1 + nprops]
        (slab, bufs, acc_sh,
         isem, gsem, ssem, csem) = refs[2 * nprops + 1 + nprops:]
        cid = lax.axis_index("c")
        sid = lax.axis_index("s")
        w = cid * NS + sid
        base = w * NCHK + jnp.minimum(w, XW)

        for p in range(nprops):
            sup = s_refs[p]
            e3 = e_refs[p]

            def _g(sl, pos, bp, sup=sup):
                pltpu.async_copy(sup.at[slab.at[sl, pos, 0]], bufs.at[bp],
                                 gsem)

            def _wg(bp, sup=sup):
                pltpu.make_async_copy(sup.at[slab.at[0, 0, 0]],
                                      bufs.at[bp], gsem).wait()

            def _s(sl, pos, bp):
                pltpu.async_copy(bufs.at[bp],
                                 acc_sh.at[slab.at[sl, pos, 1]],
                                 ssem, add=True)

            def _ds():
                pltpu.make_async_copy(bufs.at[0],
                                      acc_sh.at[slab.at[0, 0, 1]],
                                      ssem).wait()

            def _idxq(k, sl, e3=e3):
                pltpu.async_copy(e3.at[pl.ds(base + 4 * k, 4)],
                                 slab.at[sl], isem)

            def _wi(e3=e3):
                pltpu.make_async_copy(e3.at[pl.ds(base, 4)],
                                      slab.at[0], isem).wait()

            pltpu.sync_copy(e3.at[pl.ds(base, 4)], slab.at[0])
            _idxq(1, 1)
            if p > 0:
                pltpu.make_async_copy(
                    acc_sh.at[pl.ds(sid * TROWS, TROWS)],
                    o_refs[p - 1].at[cid, pl.ds(sid * TROWS, TROWS)],
                    csem).wait()
            pltpu.sync_copy(z_ref.at[pl.ds(sid * TROWS, TROWS)],
                            acc_sh.at[pl.ds(sid * TROWS, TROWS)])
            _g(0, 0, 0)
            plsc.subcore_barrier()

            _g(0, 1, 1)
            _wg(0)
            _s(0, 0, 0)
            _ds(); _g(0, 2, 0); _wg(1); _s(0, 1, 1)
            _ds(); _g(0, 3, 1); _wg(0); _s(0, 2, 0)
            _wi()
            _idxq(2, 2)
            _ds(); _g(1, 0, 0); _wg(1); _s(0, 3, 1)

            def steady(g, carry):
                hs = g % 3
                hn = (g + 1) % 3
                hl = (g + 2) % 3
                _ds(); _g(hs, 1, 1); _wg(0); _s(hs, 0, 0)
                _ds(); _g(hs, 2, 0); _wg(1); _s(hs, 1, 1)
                _ds(); _g(hs, 3, 1); _wg(0); _s(hs, 2, 0)
                _wi()
                _idxq(g + 2, hl)
                _ds(); _g(hn, 0, 0); _wg(1); _s(hs, 3, 1)
                return carry

            lax.fori_loop(1, 17, steady, 0)

            _ds(); _g(2, 1, 1); _wg(0); _s(2, 0, 0)
            _ds(); _g(2, 2, 0); _wg(1); _s(2, 1, 1)
            _ds(); _g(2, 3, 1); _wg(0); _s(2, 2, 0)
            _wi()
            _ds(); _g(0, 0, 0); _wg(1); _s(2, 3, 1)

            pltpu.sync_copy(e3.at[pl.ds(base + 76, 2)],
                            slab.at[1, pl.ds(0, 2)])
            exc = jnp.minimum(base + NCHK, TOTCH - 1)
            pltpu.sync_copy(e3.at[pl.ds(exc, 1)],
                            slab.at[1, pl.ds(2, 1)])

            _ds(); _g(0, 1, 1); _wg(0); _s(0, 0, 0)
            _ds(); _g(0, 2, 0); _wg(1); _s(0, 1, 1)
            _ds(); _g(0, 3, 1); _wg(0); _s(0, 2, 0)
            _ds(); _g(1, 0, 0); _wg(1); _s(0, 3, 1)

            _ds(); _g(1, 1, 1); _wg(0); _s(1, 0, 0)
            _ds(); _wg(1); _s(1, 1, 1)

            @pl.when(w < XW)
            def _extra():
                _g(1, 2, 0); _wg(0); _s(1, 2, 0)

            _ds()

            @pl.when(w < XW)
            def _extra_drain():
                _ds()

            plsc.subcore_barrier()
            if p < nprops - 1:
                pltpu.async_copy(
                    acc_sh.at[pl.ds(sid * TROWS, TROWS)],
                    o_refs[p].at[cid, pl.ds(sid * TROWS, TROWS)], csem)
            else:
                pltpu.sync_copy(
                    acc_sh.at[pl.ds(sid * TROWS, TROWS)],
                    o_refs[p].at[cid, pl.ds(sid * TROWS, TROWS)])

    kern = pl.kernel(
        body,
        out_type=tuple(jax.ShapeDtypeStruct((NC, NP, D), jnp.float32)
                       for _ in range(nprops)),
        mesh=mesh,
        scratch_types=[
            pltpu.VMEM((3, 4, 2, CHUNK), jnp.int32),
            pltpu.VMEM((2, CHUNK, D), jnp.float32),
            pltpu.VMEM_SHARED((NP, D), jnp.float32),
            pltpu.SemaphoreType.DMA,
            pltpu.SemaphoreType.DMA,
            pltpu.SemaphoreType.DMA,
            pltpu.SemaphoreType.DMA,
        ],
    )
    return kern(*edges, *sups, zeros)


def _sc_stage(edge_list, sup_list, zeros):
    outs = _sc_stage_call(len(edge_list), tuple(edge_list), tuple(sup_list),
                          zeros)
    return list(outs)


_ROW = pl.BlockSpec((BN, D), lambda j: (j, 0))
_PART = pl.BlockSpec((NC, BN, D), lambda j: (0, j, 0))
_W = pl.BlockSpec((D, D), lambda j: (0, 0))
_W2 = pl.BlockSpec((2 * D, D), lambda j: (0, 0))
_B = pl.BlockSpec((1, D), lambda j: (0, 0))

_sds = lambda: jax.ShapeDtypeStruct((NP, D), jnp.float32)


def _tc1(su, tu, w1, w2):
    def body(su_r, tu_r, w1_r, w2_r, oa_r, oc_r):
        oa_r[...] = _mm(su_r[...], w1_r[...])
        oc_r[...] = _mm(tu_r[...], w2_r[...])

    return pl.pallas_call(
        body, grid=(GRID,),
        in_specs=[_ROW, _ROW, _W, _W],
        out_specs=[_ROW, _ROW],
        out_shape=[_sds(), _sds()],
    )(su, tu, w1, w2)


def _tc2(pa, pc, b1, b2, w3, w4):
    def body(pa_r, pc_r, b1_r, b2_r, w3_r, w4_r, ob_r, od_r):
        s_h1 = _lrelu(pa_r[0] + pa_r[1] + b1_r[...])
        t_h1 = _lrelu(pc_r[0] + pc_r[1] + b2_r[...])
        ob_r[...] = _mm(s_h1, w3_r[...])
        od_r[...] = _mm(t_h1, w4_r[...])

    return pl.pallas_call(
        body, grid=(GRID,),
        in_specs=[_PART, _PART, _B, _B, _W, _W],
        out_specs=[_ROW, _ROW],
        out_shape=[_sds(), _sds()],
    )(pa, pc, b1.reshape(1, D), b2.reshape(1, D), w3, w4)


def _tc3(pb, pd, b3, b4, su, tu, su_w, su_b, tu_w, tu_b, we, wh):
    def body(pb_r, pd_r, b3_r, b4_r, su_r, tu_r, suw_r, sub_r, tuw_r,
             tub_r, we_r, wh_r, ou_r, oe_r, oh_r):
        s_h2 = _lrelu(pb_r[0] + pb_r[1] + b3_r[...])
        t_h2 = _lrelu(pd_r[0] + pd_r[1] + b4_r[...])
        s_user = (_mm(s_h2, suw_r[:D]) + _mm(su_r[...], suw_r[D:])
                  + sub_r[...])
        t_user = (_mm(t_h2, tuw_r[:D]) + _mm(tu_r[...], tuw_r[D:])
                  + tub_r[...])
        u = (RATE * jnp.maximum(s_user, 0.0)
             + (1.0 - RATE) * jnp.maximum(t_user, 0.0))
        ou_r[...] = u
        oe_r[...] = _mm(u, we_r[...])
        oh_r[...] = _mm(u, wh_r[...])

    return pl.pallas_call(
        body, grid=(GRID,),
        in_specs=[_PART, _PART, _B, _B, _ROW, _ROW, _W2, _B, _W2, _B,
                  _W, _W],
        out_specs=[_ROW, _ROW, _ROW],
        out_shape=[_sds(), _sds(), _sds()],
    )(pb, pd, b3.reshape(1, D), b4.reshape(1, D), su, tu, su_w,
      su_b.reshape(1, D), tu_w, tu_b.reshape(1, D), we, wh)


def _tc4(ps, pt, b1, b2, w3m, w3s, w4m, w4s):
    def body(ps_r, pt_r, b1_r, b2_r, w3m_r, w3s_r, w4m_r, w4s_r,
             of_r, og_r, oi_r, oj_r):
        s_g1 = _lrelu(ps_r[0] + ps_r[1] + b1_r[...])
        t_g1 = _lrelu(pt_r[0] + pt_r[1] + b2_r[...])
        of_r[...] = _mm(s_g1, w3m_r[...])
        og_r[...] = _mm(s_g1, w3s_r[...])
        oi_r[...] = _mm(t_g1, w4m_r[...])
        oj_r[...] = _mm(t_g1, w4s_r[...])

    return pl.pallas_call(
        body, grid=(GRID,),
        in_specs=[_PART, _PART, _B, _B, _W, _W, _W, _W],
        out_specs=[_ROW, _ROW, _ROW, _ROW],
        out_shape=[_sds(), _sds(), _sds(), _sds()],
    )(ps, pt, b1.reshape(1, D), b2.reshape(1, D), w3m, w3s, w4m, w4s)


def _tc5(pf, pg, pi, pj, bm, bs, cm, cs, u,
         sum_w, sum_b, sus_w, sus_b, tum_w, tum_b, tus_w, tus_b):
    def body(pf_r, pg_r, pi_r, pj_r, bm_r, bs_r, cm_r, cs_r, u_r,
             sumw_r, sumb_r, susw_r, susb_r, tumw_r, tumb_r, tusw_r,
             tusb_r, om_r, os_r):
        s_m = _lrelu(pf_r[0] + pf_r[1] + bm_r[...])
        s_s = _lrelu(pg_r[0] + pg_r[1] + bs_r[...])
        t_m = _lrelu(pi_r[0] + pi_r[1] + cm_r[...])
        t_s = _lrelu(pj_r[0] + pj_r[1] + cs_r[...])
        u = u_r[...]
        sm = _mm(s_m, sumw_r[:D]) + _mm(u, sumw_r[D:]) + sumb_r[...]
        ss = _mm(s_s, susw_r[:D]) + _mm(u, susw_r[D:]) + susb_r[...]
        tm = _mm(t_m, tumw_r[:D]) + _mm(u, tumw_r[D:]) + tumb_r[...]
        ts = _mm(t_s, tusw_r[:D]) + _mm(u, tusw_r[D:]) + tusb_r[...]
        om_r[...] = RATE * sm + (1.0 - RATE) * tm
        os_r[...] = RATE * ss + (1.0 - RATE) * ts

    rowN = pl.BlockSpec((N // GRID, D), lambda j: (j, 0))
    partN = pl.BlockSpec((NC, N // GRID, D), lambda j: (0, j, 0))
    sdsN = jax.ShapeDtypeStruct((N, D), jnp.float32)
    return pl.pallas_call(
        body, grid=(GRID,),
        in_specs=[partN, partN, partN, partN, _B, _B, _B, _B, rowN,
                  _W2, _B, _W2, _B, _W2, _B, _W2, _B],
        out_specs=[rowN, rowN],
        out_shape=[sdsN, sdsN],
    )(pf, pg, pi, pj, bm.reshape(1, D), bs.reshape(1, D),
      cm.reshape(1, D), cs.reshape(1, D), u,
      sum_w, sum_b.reshape(1, D), sus_w, sus_b.reshape(1, D),
      tum_w, tum_b.reshape(1, D), tus_w, tus_b.reshape(1, D))


def kernel(source_ufea, target_ufea, source_UV_edge_index,
           source_VU_edge_index, target_UV_edge_index, target_VU_edge_index,
           l1_gc1_W, l1_gc1_b, l1_gc2_W, l1_gc2_b, l1_gc3_W, l1_gc3_b,
           l1_gc4_W, l1_gc4_b, l1_su_W, l1_su_b, l1_tu_W, l1_tu_b,
           l2_gc1_W, l2_gc1_b, l2_gc2_W, l2_gc2_b, l2_gc3m_W, l2_gc3m_b,
           l2_gc3s_W, l2_gc3s_b, l2_gc4m_W, l2_gc4m_b, l2_gc4s_W, l2_gc4s_b,
           l2_sum_W, l2_sum_b, l2_sus_W, l2_sus_b, l2_tum_W, l2_tum_b,
           l2_tus_W, l2_tus_b):
    su = jnp.pad(source_ufea, ((0, NP - N), (0, 0)))
    tu = jnp.pad(target_ufea, ((0, NP - N), (0, 0)))
    def _chunked(e):
        return e.reshape(2, TOTCH, CHUNK).transpose(1, 0, 2)

    e_suv = _chunked(source_UV_edge_index)
    e_svu = _chunked(source_VU_edge_index)
    e_tuv = _chunked(target_UV_edge_index)
    e_tvu = _chunked(target_VU_edge_index)
    zeros = jnp.zeros((NP, D), jnp.float32)

    sup_a, sup_c = _tc1(su, tu, l1_gc1_W, l1_gc2_W)
    part_a, part_c = _sc_stage([e_svu, e_tvu], [sup_a, sup_c], zeros)
    sup_b, sup_d = _tc2(part_a, part_c, l1_gc1_b, l1_gc2_b,
                        l1_gc3_W, l1_gc4_W)
    part_b, part_d = _sc_stage([e_suv, e_tuv], [sup_b, sup_d], zeros)
    u, sup_e, sup_h = _tc3(part_b, part_d, l1_gc3_b, l1_gc4_b, su, tu,
                           l1_su_W, l1_su_b, l1_tu_W, l1_tu_b,
                           l2_gc1_W, l2_gc2_W)
    part_e, part_h = _sc_stage([e_svu, e_tvu], [sup_e, sup_h], zeros)
    sup_f, sup_g, sup_i, sup_j = _tc4(part_e, part_h, l2_gc1_b, l2_gc2_b,
                                      l2_gc3m_W, l2_gc3s_W,
                                      l2_gc4m_W, l2_gc4s_W)
    part_f, part_g, part_i, part_j = _sc_stage(
        [e_suv, e_suv, e_tuv, e_tuv], [sup_f, sup_g, sup_i, sup_j], zeros)
    mean, sigma = _tc5(part_f, part_g, part_i, part_j,
                       l2_gc3m_b, l2_gc3s_b, l2_gc4m_b, l2_gc4s_b, u,
                       l2_sum_W, l2_sum_b, l2_sus_W, l2_sus_b,
                       l2_tum_W, l2_tum_b, l2_tus_W, l2_tus_b)
    return (mean, sigma)
